# baseline (speedup 1.0000x reference)
import jax
import jax.numpy as jnp
from jax.experimental import pallas as pl
from jax.experimental.pallas import tpu as pltpu

_N = 131
_LATDIM = 512
_HYPERNUM = 512
_GNN_LAYER = 2

_CONTRACT_LANES = (((1,), (1,)), ((), ()))


def _fused_kernel(adj_v, u_v, i_v, uh_v, ih_v,
                  out_h, gnn_h, hyp_h,
                  out_v, gnn_v, hyp_v,
                  out_sems):
    f32 = jnp.float32
    u = u_v[...]
    i = i_v[...]
    adj = adj_v[...]
    embeds = u + i

    tem0 = jnp.dot(adj, embeds, preferred_element_type=f32)
    gnn_v[0] = tem0
    cp_gnn0 = pltpu.make_async_copy(gnn_v.at[0], gnn_h.at[0], out_sems.at[0])
    cp_gnn0.start()

    uu = jnp.dot(u, uh_v[...], preferred_element_type=f32)
    gu = jax.lax.dot_general(uu, uu, _CONTRACT_LANES, preferred_element_type=f32)
    ii = jnp.dot(i, ih_v[...], preferred_element_type=f32)
    g = gu + jax.lax.dot_general(ii, ii, _CONTRACT_LANES, preferred_element_type=f32)

    h0 = jnp.dot(g, embeds, preferred_element_type=f32)
    hyp_v[0] = h0
    cp_hyp0 = pltpu.make_async_copy(hyp_v.at[0], hyp_h.at[0], out_sems.at[1])
    cp_hyp0.start()

    lat1 = tem0 + h0
    tem1 = jnp.dot(adj, lat1, preferred_element_type=f32)
    gnn_v[1] = tem1
    cp_gnn1 = pltpu.make_async_copy(gnn_v.at[1], gnn_h.at[1], out_sems.at[2])
    cp_gnn1.start()
    h1 = jnp.dot(g, lat1, preferred_element_type=f32)
    hyp_v[1] = h1
    cp_hyp1 = pltpu.make_async_copy(hyp_v.at[1], hyp_h.at[1], out_sems.at[3])
    cp_hyp1.start()

    out_v[...] = 0.0101 * (embeds + lat1 + (tem1 + h1))
    cp_out = pltpu.make_async_copy(out_v, out_h, out_sems.at[4])
    cp_out.start()

    cp_gnn0.wait()
    cp_hyp0.wait()
    cp_gnn1.wait()
    cp_hyp1.wait()
    cp_out.wait()


def kernel(adj, uEmbeds, iEmbeds, uHyper, iHyper):
    f32 = jnp.float32
    hbm = pl.BlockSpec(memory_space=pltpu.MemorySpace.HBM)
    out_shapes = (
        jax.ShapeDtypeStruct((_N, _LATDIM), f32),
        jax.ShapeDtypeStruct((_GNN_LAYER, _N, _LATDIM), f32),
        jax.ShapeDtypeStruct((_GNN_LAYER, _N, _LATDIM), f32),
    )
    return pl.pallas_call(
        _fused_kernel,
        out_specs=(hbm, hbm, hbm),
        out_shape=out_shapes,
        scratch_shapes=[
            pltpu.VMEM((_N, _LATDIM), f32),
            pltpu.VMEM((_GNN_LAYER, _N, _LATDIM), f32),
            pltpu.VMEM((_GNN_LAYER, _N, _LATDIM), f32),
            pltpu.SemaphoreType.DMA((5,)),
        ],
    )(adj, uEmbeds, iEmbeds, uHyper, iHyper)
